# Initial kernel scaffold; baseline (speedup 1.0000x reference)
import jax
import jax.numpy as jnp
from jax.experimental import pallas as pl


def kernel(idx0, idx1, idx2, ent_embed, rule_w, W, b):
    raise NotImplementedError("write your pallas kernel here")


# speedup vs baseline: 5.0471x; 5.0471x over previous
import functools

import jax
import jax.numpy as jnp
from jax import lax
from jax.experimental import pallas as pl
from jax.experimental.pallas import tpu as pltpu
from jax.experimental.pallas import tpu_sc as plsc

B, R, N, DIM, E = 1024, 8, 8, 64, 100000

NC, NS = 2, 16
NW = NC * NS
N0 = (B * R) // NW
N1 = (B * R * N) // NW
N2 = (B * R * N * N) // NW
CH = 512
G = CH // N
LC = DIM // 16


def _sc_gather(table, i0, i1, i2):
    mesh = plsc.VectorSubcoreMesh(core_axis_name="c", subcore_axis_name="s")

    @functools.partial(
        pl.kernel,
        out_type=(
            jax.ShapeDtypeStruct((B * R, DIM), jnp.float32),
            jax.ShapeDtypeStruct((B * R * N, DIM), jnp.float32),
            jax.ShapeDtypeStruct((B * R * N, DIM), jnp.float32),
        ),
        mesh=mesh,
        scratch_types=[
            pltpu.VMEM((CH,), jnp.int32),
            pltpu.VMEM((CH, DIM), jnp.float32),
            pltpu.VMEM((G, DIM), jnp.float32),
            pltpu.SemaphoreType.DMA,
        ],
        compiler_params=pltpu.CompilerParams(use_tc_tiling_on_sc=False),
    )
    def k(table_h, i0_h, i1_h, i2_h, e0_h, e1_h, s2_h, idx_v, rows_v, acc_v, sem):
        wid = lax.axis_index("s") * NC + lax.axis_index("c")
        b0 = wid * N0
        pltpu.sync_copy(i0_h.at[pl.ds(b0, N0)], idx_v.at[pl.ds(0, N0)])
        pltpu.async_copy(table_h.at[idx_v.at[pl.ds(0, N0)]],
                         rows_v.at[pl.ds(0, N0)], sem).wait()
        pltpu.sync_copy(rows_v.at[pl.ds(0, N0)], e0_h.at[pl.ds(b0, N0)])

        for c in range(N1 // CH):
            b1 = wid * N1 + c * CH
            pltpu.sync_copy(i1_h.at[pl.ds(b1, CH)], idx_v)
            pltpu.async_copy(table_h.at[idx_v], rows_v, sem).wait()
            pltpu.sync_copy(rows_v, e1_h.at[pl.ds(b1, CH)])

        def chunk(c, carry):
            b2 = wid * N2 + c * CH
            pltpu.sync_copy(i2_h.at[pl.ds(b2, CH)], idx_v)
            pltpu.async_copy(table_h.at[idx_v], rows_v, sem).wait()

            def grp(g, carry2):
                base = g * N
                for lc in range(LC):
                    acc = rows_v[base, pl.ds(lc * 16, 16)]
                    for kk in range(1, N):
                        acc = acc + rows_v[base + kk, pl.ds(lc * 16, 16)]
                    acc_v[g, pl.ds(lc * 16, 16)] = acc
                return carry2

            lax.fori_loop(0, G, grp, 0)
            pltpu.sync_copy(acc_v, s2_h.at[pl.ds(wid * (N2 // N) + c * G, G)])
            return carry

        lax.fori_loop(0, N2 // CH, chunk, 0)

    return k(table, i0, i1, i2)


NB = 256


def _tc_compute(e0, e1, s2, Wm, bias_row, rule_rows):
    inv = 1.0 / N

    def body(e0_ref, e1_ref, s2_ref, w_ref, b_ref, rw_ref, out_ref):
        Wf = w_ref[...]
        bb = b_ref[...]
        tot = jnp.zeros((NB, DIM), jnp.float32)
        for r in range(R):
            m0 = e1_ref[0 * R + r]
            for kk in range(1, N):
                m0 = m0 + e1_ref[kk * R + r]
            m0 = m0 * inv
            m1 = jnp.zeros((NB, DIM), jnp.float32)
            for kk in range(N):
                l = kk * R + r
                x = jnp.concatenate([e1_ref[l], s2_ref[l] * inv], axis=-1)
                h1 = jax.nn.relu(
                    lax.dot(x, Wf, precision=lax.Precision.HIGHEST) + bb)
                m1 = m1 + h1
            m1 = m1 * inv
            x0 = jnp.concatenate([e0_ref[r], m0], axis=-1)
            h0 = jax.nn.relu(
                lax.dot(x0, Wf, precision=lax.Precision.HIGHEST) + bb)
            xo = jnp.concatenate([h0, m1], axis=-1)
            o = jnp.tanh(
                lax.dot(xo, Wf, precision=lax.Precision.HIGHEST) + bb)
            tot = tot + o * rw_ref[pl.ds(r, 1), :]
        out_ref[...] = tot

    return pl.pallas_call(
        body,
        grid=(B // NB,),
        in_specs=[
            pl.BlockSpec((R, NB, DIM), lambda i: (0, i, 0)),
            pl.BlockSpec((R * N, NB, DIM), lambda i: (0, i, 0)),
            pl.BlockSpec((R * N, NB, DIM), lambda i: (0, i, 0)),
            pl.BlockSpec((2 * DIM, DIM), lambda i: (0, 0)),
            pl.BlockSpec((1, DIM), lambda i: (0, 0)),
            pl.BlockSpec((R, DIM), lambda i: (0, 0)),
        ],
        out_specs=pl.BlockSpec((NB, DIM), lambda i: (i, 0)),
        out_shape=jax.ShapeDtypeStruct((B, DIM), jnp.float32),
    )(e0, e1, s2, Wm, bias_row, rule_rows)


def kernel(idx0, idx1, idx2, ent_embed, rule_w, W, b):
    i0 = idx0.astype(jnp.int32).T.reshape(-1)
    i1 = idx1.astype(jnp.int32).reshape(B, R, N).transpose(2, 1, 0).reshape(-1)
    i2 = idx2.astype(jnp.int32).reshape(B, R, N, N).transpose(2, 1, 0, 3).reshape(-1)
    e0, e1, s2 = _sc_gather(ent_embed, i0, i1, i2)
    e0 = e0.reshape(R, B, DIM)
    e1 = e1.reshape(R * N, B, DIM)
    s2 = s2.reshape(R * N, B, DIM)
    bias_row = b.reshape(1, DIM)
    rule_rows = jnp.broadcast_to(rule_w.reshape(R, 1), (R, DIM))
    return _tc_compute(e0, e1, s2, W, bias_row, rule_rows)
